# baseline (speedup 1.0000x reference)
import functools

import jax
import jax.numpy as jnp
from jax.experimental import pallas as pl
from jax.experimental.pallas import tpu as pltpu

_B, _N, _K, _FC, _FN = 4, 10000, 16, 32, 128


def _node_logits_body(x_ref, w1_ref, b1_ref, w2_ref, b2_ref, wfc_ref, bfc_ref,
                      out_ref):
    x = x_ref[0]
    h = jnp.maximum(
        jnp.dot(x, w1_ref[...], preferred_element_type=jnp.float32)
        + b1_ref[...], 0.0)
    h = jnp.maximum(
        jnp.dot(h, w2_ref[...], preferred_element_type=jnp.float32)
        + b2_ref[...], 0.0)
    out_ref[0] = (
        jnp.dot(h, wfc_ref[...], preferred_element_type=jnp.float32)
        + bfc_ref[...])


def _col_out_body(lgrow_ref, lgcol_ref, colx_ref, w1k_ref, b1k_ref, w2k_ref,
                  bc2_ref, out_ref):
    row = lgrow_ref[0]
    m = jnp.max(row)
    s = jnp.sum(jnp.exp(row - m))
    nodep = jnp.exp(lgcol_ref[0] - m) / s

    x4 = colx_ref[0]
    x = x4.reshape(x4.shape[0], x4.shape[1] * x4.shape[2])
    h = jnp.maximum(
        jnp.dot(x, w1k_ref[...], preferred_element_type=jnp.float32)
        + b1k_ref[...], 0.0)
    cl = (jnp.dot(h, w2k_ref[...], preferred_element_type=jnp.float32)
          + bc2_ref[0, 0])
    cm = jnp.max(cl, axis=1, keepdims=True)
    ce = jnp.exp(cl - cm)
    cp = ce / jnp.sum(ce, axis=1, keepdims=True)
    out_ref[0] = cp * nodep


@jax.jit
def kernel(node_features, col_features, edge_index, W1, b1, W2, b2, Wfc, bfc,
           Wc1, bc1, Wc2, bc2):
    del edge_index
    B, N, FN = node_features.shape
    K, FC = col_features.shape[2], col_features.shape[3]
    H1 = W1.shape[1]

    bn = 2000
    grid = (B, N // bn)

    x3 = node_features
    logits = pl.pallas_call(
        _node_logits_body,
        grid=grid,
        in_specs=[
            pl.BlockSpec((1, bn, FN), lambda b, i: (b, i, 0)),
            pl.BlockSpec((FN, H1), lambda b, i: (0, 0)),
            pl.BlockSpec((1, H1), lambda b, i: (0, 0)),
            pl.BlockSpec((H1, H1), lambda b, i: (0, 0)),
            pl.BlockSpec((1, H1), lambda b, i: (0, 0)),
            pl.BlockSpec((H1, 1), lambda b, i: (0, 0)),
            pl.BlockSpec((1, 1), lambda b, i: (0, 0)),
        ],
        out_specs=pl.BlockSpec((1, bn, 1), lambda b, i: (b, i, 0)),
        out_shape=jax.ShapeDtypeStruct((B, N, 1), jnp.float32),
    )(x3, W1, b1.reshape(1, -1), W2, b2.reshape(1, -1), Wfc,
      bfc.reshape(1, 1))

    eye = jnp.eye(K, dtype=jnp.float32)
    W1k = jnp.kron(eye, Wc1)
    W2k = jnp.kron(eye, Wc2)
    b1k = jnp.tile(bc1, K).reshape(1, -1)

    lgrow = logits.reshape(B, 1, N)

    out = pl.pallas_call(
        _col_out_body,
        grid=grid,
        in_specs=[
            pl.BlockSpec((1, 1, N), lambda b, i: (b, 0, 0)),
            pl.BlockSpec((1, bn, 1), lambda b, i: (b, i, 0)),
            pl.BlockSpec((1, bn, K, FC), lambda b, i: (b, i, 0, 0)),
            pl.BlockSpec((K * FC, K * 16), lambda b, i: (0, 0)),
            pl.BlockSpec((1, K * 16), lambda b, i: (0, 0)),
            pl.BlockSpec((K * 16, K), lambda b, i: (0, 0)),
            pl.BlockSpec((1, 1), lambda b, i: (0, 0)),
        ],
        out_specs=pl.BlockSpec((1, bn, K), lambda b, i: (b, i, 0)),
        out_shape=jax.ShapeDtypeStruct((B, N, K), jnp.float32),
    )(lgrow, logits, col_features, W1k, b1k, W2k, bc2.reshape(1, 1))

    return out.reshape(B, N * K)


# speedup vs baseline: 3.8968x; 3.8968x over previous
import jax
import jax.numpy as jnp
from jax.experimental import pallas as pl
from jax.experimental.pallas import tpu as pltpu


def _node_logits_body(x_ref, w1_ref, b1_ref, w2_ref, b2_ref, wfc_ref, bfc_ref,
                      out_ref):
    x = x_ref[0]
    h = jnp.maximum(
        jnp.dot(x, w1_ref[...], preferred_element_type=jnp.float32)
        + b1_ref[...], 0.0)
    h = jnp.maximum(
        jnp.dot(h, w2_ref[...], preferred_element_type=jnp.float32)
        + b2_ref[...], 0.0)
    out_ref[0] = (
        jnp.dot(h, wfc_ref[...], preferred_element_type=jnp.float32)
        + bfc_ref[...])


def _col_out_body(lgrow_ref, colx_ref, w4_ref, b4_ref, w24_ref, bc2_ref,
                  out_ref, cl_ref):
    j = pl.program_id(1)
    n = colx_ref.shape[3]
    x = colx_ref[0]
    for g in range(2):
        xg = x[4 * g:4 * (g + 1)].reshape(4 * 32, n)
        hg = jnp.maximum(
            jnp.dot(w4_ref[...], xg, preferred_element_type=jnp.float32)
            + b4_ref[...], 0.0)
        clg = (jnp.dot(w24_ref[...], hg, preferred_element_type=jnp.float32)
               + bc2_ref[0, 0])
        cl_ref[j, 4 * g:4 * (g + 1), :] = clg

    @pl.when(j == 1)
    def _finish():
        row = lgrow_ref[0]
        m = jnp.max(row)
        e = jnp.exp(row - m)
        nodep = e / jnp.sum(e)
        cl = cl_ref[...].reshape(16, n)
        cm = jnp.max(cl, axis=0, keepdims=True)
        ce = jnp.exp(cl - cm)
        cp = ce / jnp.sum(ce, axis=0, keepdims=True)
        out_ref[0] = cp * nodep


@jax.jit
def kernel(node_features, col_features, edge_index, W1, b1, W2, b2, Wfc, bfc,
           Wc1, bc1, Wc2, bc2):
    del edge_index
    B, N, FN = node_features.shape
    K, FC = col_features.shape[2], col_features.shape[3]
    H1 = W1.shape[1]

    bn = 2000

    logits = pl.pallas_call(
        _node_logits_body,
        grid=(B, N // bn),
        in_specs=[
            pl.BlockSpec((1, bn, FN), lambda b, i: (b, i, 0)),
            pl.BlockSpec((FN, H1), lambda b, i: (0, 0)),
            pl.BlockSpec((1, H1), lambda b, i: (0, 0)),
            pl.BlockSpec((H1, H1), lambda b, i: (0, 0)),
            pl.BlockSpec((1, H1), lambda b, i: (0, 0)),
            pl.BlockSpec((H1, 1), lambda b, i: (0, 0)),
            pl.BlockSpec((1, 1), lambda b, i: (0, 0)),
        ],
        out_specs=pl.BlockSpec((1, bn, 1), lambda b, i: (b, i, 0)),
        out_shape=jax.ShapeDtypeStruct((B, N, 1), jnp.float32),
    )(node_features, W1, b1.reshape(1, -1), W2, b2.reshape(1, -1), Wfc,
      bfc.reshape(1, 1))

    eye4 = jnp.eye(4, dtype=jnp.float32)
    W4 = jnp.kron(eye4, Wc1.T)
    W24 = jnp.kron(eye4, Wc2.T)
    b4 = jnp.tile(bc1, 4).reshape(-1, 1)

    colT = jnp.transpose(col_features, (0, 2, 3, 1))
    lgrow = logits.reshape(B, 1, N)

    outT = pl.pallas_call(
        _col_out_body,
        grid=(B, 2),
        in_specs=[
            pl.BlockSpec((1, 1, N), lambda b, j: (b, 0, 0)),
            pl.BlockSpec((1, K // 2, FC, N), lambda b, j: (b, j, 0, 0)),
            pl.BlockSpec((64, 128), lambda b, j: (0, 0)),
            pl.BlockSpec((64, 1), lambda b, j: (0, 0)),
            pl.BlockSpec((4, 64), lambda b, j: (0, 0)),
            pl.BlockSpec((1, 1), lambda b, j: (0, 0)),
        ],
        out_specs=pl.BlockSpec((1, K, N), lambda b, j: (b, 0, 0)),
        out_shape=jax.ShapeDtypeStruct((B, K, N), jnp.float32),
        scratch_shapes=[pltpu.VMEM((2, K // 2, N), jnp.float32)],
    )(lgrow, colT, W4, b4, W24, bc2.reshape(1, 1))

    return jnp.swapaxes(outT, 1, 2).reshape(B, N * K)


# speedup vs baseline: 4.2915x; 1.1013x over previous
import jax
import jax.numpy as jnp
from jax.experimental import pallas as pl
from jax.experimental.pallas import tpu as pltpu


def _node_logits_body(x_ref, w1_ref, b1_ref, w2_ref, b2_ref, wfc_ref, bfc_ref,
                      out_ref):
    x = x_ref[0]
    h = jnp.maximum(
        jnp.dot(x, w1_ref[...], preferred_element_type=jnp.float32)
        + b1_ref[...], 0.0)
    h = jnp.maximum(
        jnp.dot(h, w2_ref[...], preferred_element_type=jnp.float32)
        + b2_ref[...], 0.0)
    lg = (jnp.dot(h, wfc_ref[...], preferred_element_type=jnp.float32)
          + bfc_ref[...])
    out_ref[0] = lg.reshape(1, lg.shape[0])


def _col_out_body(lgrow_ref, colx_ref, w4_ref, b4_ref, w24_ref, bc2_ref,
                  out_ref, cl_ref):
    j = pl.program_id(1)
    n = colx_ref.shape[3]
    x = colx_ref[0]
    for g in range(2):
        xg = x[4 * g:4 * (g + 1)].reshape(4 * 32, n)
        hg = jnp.maximum(
            jnp.dot(w4_ref[...], xg, preferred_element_type=jnp.float32)
            + b4_ref[...], 0.0)
        clg = (jnp.dot(w24_ref[...], hg, preferred_element_type=jnp.float32)
               + bc2_ref[0, 0])
        cl_ref[j, 4 * g:4 * (g + 1), :] = clg

    @pl.when(j == 1)
    def _finish():
        row = lgrow_ref[0]
        m = jnp.max(row)
        e = jnp.exp(row - m)
        nodep = e / jnp.sum(e)
        cl = cl_ref[...].reshape(16, n)
        cm = jnp.max(cl, axis=0, keepdims=True)
        ce = jnp.exp(cl - cm)
        cp = ce / jnp.sum(ce, axis=0, keepdims=True)
        out_ref[0] = cp * nodep


@jax.jit
def kernel(node_features, col_features, edge_index, W1, b1, W2, b2, Wfc, bfc,
           Wc1, bc1, Wc2, bc2):
    del edge_index
    B, N, FN = node_features.shape
    K, FC = col_features.shape[2], col_features.shape[3]
    H1 = W1.shape[1]

    logits = pl.pallas_call(
        _node_logits_body,
        grid=(B,),
        in_specs=[
            pl.BlockSpec((1, N, FN), lambda b: (b, 0, 0)),
            pl.BlockSpec((FN, H1), lambda b: (0, 0)),
            pl.BlockSpec((1, H1), lambda b: (0, 0)),
            pl.BlockSpec((H1, H1), lambda b: (0, 0)),
            pl.BlockSpec((1, H1), lambda b: (0, 0)),
            pl.BlockSpec((H1, 1), lambda b: (0, 0)),
            pl.BlockSpec((1, 1), lambda b: (0, 0)),
        ],
        out_specs=pl.BlockSpec((1, 1, N), lambda b: (b, 0, 0)),
        out_shape=jax.ShapeDtypeStruct((B, 1, N), jnp.float32),
    )(node_features, W1, b1.reshape(1, -1), W2, b2.reshape(1, -1), Wfc,
      bfc.reshape(1, 1))

    eye4 = jnp.eye(4, dtype=jnp.float32)
    W4 = jnp.kron(eye4, Wc1.T)
    W24 = jnp.kron(eye4, Wc2.T)
    b4 = jnp.tile(bc1, 4).reshape(-1, 1)

    colT = jnp.transpose(col_features, (0, 2, 3, 1))

    out = pl.pallas_call(
        _col_out_body,
        grid=(B, 2),
        in_specs=[
            pl.BlockSpec((1, 1, N), lambda b, j: (b, 0, 0)),
            pl.BlockSpec((1, K // 2, FC, N), lambda b, j: (b, j, 0, 0)),
            pl.BlockSpec((64, 128), lambda b, j: (0, 0)),
            pl.BlockSpec((64, 1), lambda b, j: (0, 0)),
            pl.BlockSpec((4, 64), lambda b, j: (0, 0)),
            pl.BlockSpec((1, 1), lambda b, j: (0, 0)),
        ],
        out_specs=pl.BlockSpec((1, K, N), lambda b, j: (b, 0, 0)),
        out_shape=jax.ShapeDtypeStruct((B, K, N), jnp.float32),
        scratch_shapes=[pltpu.VMEM((2, K // 2, N), jnp.float32)],
    )(logits, colT, W4, b4, W24, bc2.reshape(1, 1))

    return jnp.swapaxes(out, 1, 2).reshape(B, N * K)


# speedup vs baseline: 6.2237x; 1.4502x over previous
import jax
import jax.numpy as jnp
from jax.experimental import pallas as pl
from jax.experimental.pallas import tpu as pltpu


def _node_logits_body(x_ref, w1_ref, b1_ref, w2_ref, b2_ref, wfc_ref, bfc_ref,
                      out_ref):
    x = x_ref[0]
    h = jnp.maximum(
        jnp.dot(x, w1_ref[...], preferred_element_type=jnp.float32)
        + b1_ref[...], 0.0)
    h = jnp.maximum(
        jnp.dot(h, w2_ref[...], preferred_element_type=jnp.float32)
        + b2_ref[...], 0.0)
    lg = jax.lax.dot_general(
        wfc_ref[...], h, (((0,), (1,)), ((), ())),
        preferred_element_type=jnp.float32)
    out_ref[0] = lg + bfc_ref[...]


def _col_out_body(lgrow_ref, colx_ref, w4_ref, b4_ref, w24_ref, bc2_ref,
                  out_ref, cl_ref, pt_ref):
    j = pl.program_id(1)
    n = colx_ref.shape[3]
    x = colx_ref[0]
    for g in range(2):
        xg = x[4 * g:4 * (g + 1)].reshape(4 * 32, n)
        hg = jnp.maximum(
            jnp.dot(w4_ref[...], xg, preferred_element_type=jnp.float32)
            + b4_ref[...], 0.0)
        clg = (jnp.dot(w24_ref[...], hg, preferred_element_type=jnp.float32)
               + bc2_ref[0, 0])
        cl_ref[j, 4 * g:4 * (g + 1), :] = clg

    @pl.when(j == 1)
    def _finish():
        row = lgrow_ref[0]
        m = jnp.max(row)
        e = jnp.exp(row - m)
        nodep = e / jnp.sum(e)
        cl = cl_ref[...].reshape(16, n)
        cm = jnp.max(cl, axis=0, keepdims=True)
        ce = jnp.exp(cl - cm)
        cp = ce / jnp.sum(ce, axis=0, keepdims=True)
        prod = cp * nodep
        pt_ref[...] = prod.T
        for r in range(8):
            out_ref[0, :, r * 16:(r + 1) * 16] = pt_ref[r::8, :]


@jax.jit
def kernel(node_features, col_features, edge_index, W1, b1, W2, b2, Wfc, bfc,
           Wc1, bc1, Wc2, bc2):
    del edge_index
    B, N, FN = node_features.shape
    K, FC = col_features.shape[2], col_features.shape[3]
    H1 = W1.shape[1]

    logits = pl.pallas_call(
        _node_logits_body,
        grid=(B,),
        in_specs=[
            pl.BlockSpec((1, N, FN), lambda b: (b, 0, 0)),
            pl.BlockSpec((FN, H1), lambda b: (0, 0)),
            pl.BlockSpec((1, H1), lambda b: (0, 0)),
            pl.BlockSpec((H1, H1), lambda b: (0, 0)),
            pl.BlockSpec((1, H1), lambda b: (0, 0)),
            pl.BlockSpec((H1, 1), lambda b: (0, 0)),
            pl.BlockSpec((1, 1), lambda b: (0, 0)),
        ],
        out_specs=pl.BlockSpec((1, 1, N), lambda b: (b, 0, 0)),
        out_shape=jax.ShapeDtypeStruct((B, 1, N), jnp.float32),
    )(node_features, W1, b1.reshape(1, -1), W2, b2.reshape(1, -1), Wfc,
      bfc.reshape(1, 1))

    eye4 = jnp.eye(4, dtype=jnp.float32)
    W4 = jnp.kron(eye4, Wc1.T)
    W24 = jnp.kron(eye4, Wc2.T)
    b4 = jnp.tile(bc1, 4).reshape(-1, 1)

    colT = jnp.transpose(col_features, (0, 2, 3, 1))

    out = pl.pallas_call(
        _col_out_body,
        grid=(B, 2),
        in_specs=[
            pl.BlockSpec((1, 1, N), lambda b, j: (b, 0, 0)),
            pl.BlockSpec((1, K // 2, FC, N), lambda b, j: (b, j, 0, 0)),
            pl.BlockSpec((64, 128), lambda b, j: (0, 0)),
            pl.BlockSpec((64, 1), lambda b, j: (0, 0)),
            pl.BlockSpec((4, 64), lambda b, j: (0, 0)),
            pl.BlockSpec((1, 1), lambda b, j: (0, 0)),
        ],
        out_specs=pl.BlockSpec((1, N // 8, 8 * K), lambda b, j: (b, 0, 0)),
        out_shape=jax.ShapeDtypeStruct((B, N // 8, 8 * K), jnp.float32),
        scratch_shapes=[
            pltpu.VMEM((2, K // 2, N), jnp.float32),
            pltpu.VMEM((N, K), jnp.float32),
        ],
    )(logits, colT, W4, b4, W24, bc2.reshape(1, 1))

    return out.reshape(B, N * K)


# speedup vs baseline: 6.5735x; 1.0562x over previous
import jax
import jax.numpy as jnp
from jax.experimental import pallas as pl
from jax.experimental.pallas import tpu as pltpu


def _fused_body(x_ref, w1t_ref, b1_ref, w2_ref, b2_ref, wfct_ref, bfc_ref,
                colx_ref, w4_ref, b4_ref, w24_ref, bc2_ref,
                out_ref, lg_ref, cl_ref, pt_ref):
    p = pl.program_id(1)
    n = colx_ref.shape[3]

    @pl.when(p == 0)
    def _node():
        x = x_ref[0]
        h = jnp.maximum(
            jax.lax.dot_general(x, w1t_ref[...], (((1,), (1,)), ((), ())),
                                preferred_element_type=jnp.float32)
            + b1_ref[...], 0.0)
        h = jnp.maximum(
            jnp.dot(h, w2_ref[...], preferred_element_type=jnp.float32)
            + b2_ref[...], 0.0)
        lg_ref[...] = (
            jax.lax.dot_general(wfct_ref[...], h, (((1,), (1,)), ((), ())),
                                preferred_element_type=jnp.float32)
            + bfc_ref[...])

    @pl.when(p > 0)
    def _col():
        j = p - 1
        x = colx_ref[0]
        for g in range(2):
            xg = x[4 * g:4 * (g + 1)].reshape(4 * 32, n)
            hg = jnp.maximum(
                jnp.dot(w4_ref[...], xg, preferred_element_type=jnp.float32)
                + b4_ref[...], 0.0)
            clg = (jnp.dot(w24_ref[...], hg,
                           preferred_element_type=jnp.float32)
                   + bc2_ref[0, 0])
            cl_ref[j, 4 * g:4 * (g + 1), :] = clg

    @pl.when(p == 2)
    def _finish():
        row = lg_ref[...]
        m = jnp.max(row)
        e = jnp.exp(row - m)
        nodep = e / jnp.sum(e)
        cl = cl_ref[...].reshape(16, n)
        cm = jnp.max(cl, axis=0, keepdims=True)
        ce = jnp.exp(cl - cm)
        cp = ce / jnp.sum(ce, axis=0, keepdims=True)
        prod = cp * nodep
        pt_ref[...] = prod.T
        for r in range(8):
            out_ref[0, :, r * 16:(r + 1) * 16] = pt_ref[r::8, :]


@jax.jit
def kernel(node_features, col_features, edge_index, W1, b1, W2, b2, Wfc, bfc,
           Wc1, bc1, Wc2, bc2):
    del edge_index
    B, N, FN = node_features.shape
    K, FC = col_features.shape[2], col_features.shape[3]
    H1 = W1.shape[1]

    eye4 = jnp.eye(4, dtype=jnp.float32)
    W4 = jnp.kron(eye4, Wc1.T)
    W24 = jnp.kron(eye4, Wc2.T)
    b4 = jnp.tile(bc1, 4).reshape(-1, 1)

    colT = jnp.transpose(col_features, (0, 2, 3, 1))

    out = pl.pallas_call(
        _fused_body,
        grid=(B, 3),
        in_specs=[
            pl.BlockSpec((1, N, FN), lambda b, p: (b, 0, 0)),
            pl.BlockSpec((H1, FN), lambda b, p: (0, 0)),
            pl.BlockSpec((1, H1), lambda b, p: (0, 0)),
            pl.BlockSpec((H1, H1), lambda b, p: (0, 0)),
            pl.BlockSpec((1, H1), lambda b, p: (0, 0)),
            pl.BlockSpec((1, H1), lambda b, p: (0, 0)),
            pl.BlockSpec((1, 1), lambda b, p: (0, 0)),
            pl.BlockSpec((1, K // 2, FC, N),
                         lambda b, p: (b, jnp.maximum(p - 1, 0), 0, 0)),
            pl.BlockSpec((64, 128), lambda b, p: (0, 0)),
            pl.BlockSpec((64, 1), lambda b, p: (0, 0)),
            pl.BlockSpec((4, 64), lambda b, p: (0, 0)),
            pl.BlockSpec((1, 1), lambda b, p: (0, 0)),
        ],
        out_specs=pl.BlockSpec((1, N // 8, 8 * K), lambda b, p: (b, 0, 0)),
        out_shape=jax.ShapeDtypeStruct((B, N // 8, 8 * K), jnp.float32),
        scratch_shapes=[
            pltpu.VMEM((1, N), jnp.float32),
            pltpu.VMEM((2, K // 2, N), jnp.float32),
            pltpu.VMEM((N, K), jnp.float32),
        ],
    )(node_features, W1.T, b1.reshape(1, -1), W2, b2.reshape(1, -1), Wfc.T,
      bfc.reshape(1, 1), colT, W4, b4, W24, bc2.reshape(1, 1))

    return out.reshape(B, N * K)
